# baseline (speedup 1.0000x reference)
import functools

import jax
import jax.numpy as jnp
from jax import lax
from jax.experimental import pallas as pl
from jax.experimental.pallas import tpu as pltpu
from jax.experimental.pallas import tpu_sc as plsc

EMB_DIM = 64
_NC = 2
_NS = 16
_NW = _NC * _NS


@functools.lru_cache(maxsize=None)
def _make_linearize(b, s):
    n_total = b * s
    n_slab = s // 8
    b_per_w = b // _NW
    j_per_w = n_total // _NW
    mesh = plsc.VectorSubcoreMesh(core_axis_name="c", subcore_axis_name="s")

    @functools.partial(
        pl.kernel,
        mesh=mesh,
        out_type=jax.ShapeDtypeStruct((n_total,), jnp.int32),
        scratch_types=[
            pltpu.VMEM((8, b), jnp.int32),
            pltpu.VMEM((8, b), jnp.int32),
            pltpu.VMEM((j_per_w,), jnp.int32),
            pltpu.SemaphoreType.DMA,
        ],
        compiler_params=pltpu.CompilerParams(
            use_tc_tiling_on_sc=True, needs_layout_passes=False),
    )
    def linearize_kernel(inp_hbm, out_hbm, slab0, slab1, stage, sem):
        wid = lax.axis_index("s") * _NC + lax.axis_index("c")
        b0 = wid * b_per_w
        slabs = [slab0, slab1]
        copies = [None] * n_slab
        copies[0] = pltpu.async_copy(inp_hbm.at[0], slabs[0], sem)
        if n_slab > 1:
            copies[1] = pltpu.async_copy(inp_hbm.at[1], slabs[1], sem)
        lane = lax.iota(jnp.int32, 16) * s
        for g in range(n_slab):
            copies[g].wait()
            slab = slabs[g % 2]
            for r in range(8):
                for c in range(b_per_w // 16):
                    v = slab[r, pl.ds(b0 + c * 16, 16)]
                    tgt = lane + (c * 16 * s + 8 * g + r)
                    plsc.store_scatter(stage, [tgt], v)
            if g + 2 < n_slab:
                copies[g + 2] = pltpu.async_copy(
                    inp_hbm.at[g + 2], slabs[g % 2], sem)
        pltpu.sync_copy(stage, out_hbm.at[pl.ds(wid * j_per_w, j_per_w)])

    return linearize_kernel


@functools.lru_cache(maxsize=None)
def _make_gather(n_total, chunk):
    b_per_w = n_total // _NW
    t = b_per_w // chunk
    mesh = plsc.VectorSubcoreMesh(core_axis_name="c", subcore_axis_name="s")

    @functools.partial(
        pl.kernel,
        mesh=mesh,
        out_type=jax.ShapeDtypeStruct((n_total, EMB_DIM), jnp.float32),
        scratch_types=[
            pltpu.VMEM((chunk,), jnp.int32),
            pltpu.VMEM((chunk,), jnp.int32),
            pltpu.VMEM((2, chunk, EMB_DIM), jnp.float32),
            pltpu.SemaphoreType.DMA,
            pltpu.SemaphoreType.DMA,
            pltpu.SemaphoreType.DMA,
        ],
        compiler_params=pltpu.CompilerParams(use_tc_tiling_on_sc=False),
    )
    def gather_kernel(idx_hbm, table_hbm, out_hbm,
                      idx_v0, idx_v1, rows_v, isem, gsem, ssem):
        wid = lax.axis_index("s") * _NC + lax.axis_index("c")
        idx_bufs = [idx_v0, idx_v1]
        ic = [None] * t
        gc = [None] * t
        sc = [None] * t

        def idx_slice(c):
            return idx_hbm.at[pl.ds((wid * t + c) * chunk, chunk)]

        ic[0] = pltpu.async_copy(idx_slice(0), idx_bufs[0], isem)
        if t > 1:
            ic[1] = pltpu.async_copy(idx_slice(1), idx_bufs[1], isem)
        ic[0].wait()
        gc[0] = pltpu.async_copy(table_hbm.at[idx_bufs[0]], rows_v.at[0], gsem)
        for c in range(t):
            p = c % 2
            if c + 1 < t:
                ic[c + 1].wait()
                if c >= 1:
                    sc[c - 1].wait()
                gc[c + 1] = pltpu.async_copy(
                    table_hbm.at[idx_bufs[(c + 1) % 2]],
                    rows_v.at[(c + 1) % 2], gsem)
            gc[c].wait()
            if c + 2 < t:
                ic[c + 2] = pltpu.async_copy(idx_slice(c + 2), idx_bufs[p], isem)
            sc[c] = pltpu.async_copy(
                rows_v.at[p],
                out_hbm.at[pl.ds((wid * t + c) * chunk, chunk)], ssem)
        if t > 1:
            sc[t - 2].wait()
        sc[t - 1].wait()

    return gather_kernel


def kernel(inp, emb_weight):
    b, s = inp.shape
    n_total = b * s
    inp_t3 = inp.T.reshape(s // 8, 8, b)
    idx_flat = _make_linearize(b, s)(inp_t3)
    out = _make_gather(n_total, 640)(idx_flat, emb_weight)
    return out.reshape(b, s, EMB_DIM)
